# baseline (device time: 27772 ns/iter reference)
import jax
import jax.numpy as jnp
from jax import lax
from jax.experimental import pallas as pl
from jax.experimental.pallas import tpu as pltpu

N_DEV = 4


def kernel(x, Win0, Wout0, Win1, Wout1, Win2, Wout2):
    m_per, d = x.shape
    B = N_DEV * m_per
    H = B // 2
    Q = H // 2

    f32 = jnp.float32
    bf16 = jnp.bfloat16

    def body(x_ref, win0, wout0, win1, wout1, win2, wout2,
             out_ref, xfull, pbuf, r1, sbuf, r2, send_sems, recv_sems):
        my_pos = lax.axis_index("i")
        pa = my_pos ^ 1
        pb = (N_DEV - 1) - my_pos
        pd = my_pos ^ 2

        barrier = pltpu.get_barrier_semaphore()
        for nbr in (pa, pb):
            pl.semaphore_signal(barrier, inc=1, device_id=(nbr,),
                                device_id_type=pl.DeviceIdType.MESH)
        pl.semaphore_wait(barrier, 2)

        pending_sends = []

        def start(src, dst, sem_idx, partner):
            rdma = pltpu.make_async_remote_copy(
                src_ref=src, dst_ref=dst,
                send_sem=send_sems.at[sem_idx],
                recv_sem=recv_sems.at[sem_idx],
                device_id=(partner,),
                device_id_type=pl.DeviceIdType.MESH,
            )
            rdma.start()
            pending_sends.append(rdma)
            return rdma

        row0 = my_pos * m_per
        xfull[pl.ds(row0, m_per), :] = x_ref[...].astype(bf16)
        my_rows = xfull.at[pl.ds(row0, m_per), :]
        for idx, peer in ((0, pa), (1, pb), (2, pd)):
            start(my_rows, my_rows, idx, peer)
        layers_bf = tuple(
            (win[...].astype(bf16), wout[...].astype(bf16))
            for win, wout in ((win0, wout0), (win1, wout1), (win2, wout2)))
        for idx, peer in ((0, pa), (1, pb), (2, pd)):
            recv = pltpu.make_async_remote_copy(
                src_ref=my_rows,
                dst_ref=xfull.at[pl.ds(peer * m_per, m_per), :],
                send_sem=send_sems.at[idx],
                recv_sem=recv_sems.at[idx],
                device_id=(peer,),
                device_id_type=pl.DeviceIdType.MESH,
            )
            recv.wait_recv()

        xtop = xfull[pl.ds(0, H), :]
        xbot = xfull[pl.ds(H, H), :]

        def partial(xh, win_bf, wout_bf):
            h = jnp.maximum(
                jnp.dot(xh, win_bf, preferred_element_type=f32), 0.0)
            return jnp.dot(h.astype(bf16), wout_bf,
                           preferred_element_type=f32)

        pend_bot = None
        for l, (win_bf, wout_bf) in enumerate(layers_bf):
            s = 3 + 8 * l
            pt = partial(xtop, win_bf, wout_bf)
            pbuf[l, 0] = pt.astype(bf16)
            a_t = [start(pbuf.at[l, 0, pl.ds(q * Q, Q)],
                         r1.at[l, 0, pl.ds(q * Q, Q)], s + q, pa)
                   for q in (0, 1)]
            if pend_bot is not None:
                cb0, cb1, lp = pend_bot
                cb0.wait_recv()
                cb1.wait_recv()
                xbot = (sbuf[lp, 1].astype(f32)
                        + r2[lp, 1].astype(f32)).astype(bf16)
            pbv = partial(xbot, win_bf, wout_bf)
            pbuf[l, 1] = pbv.astype(bf16)
            a_b = [start(pbuf.at[l, 1, pl.ds(q * Q, Q)],
                         r1.at[l, 1, pl.ds(q * Q, Q)], s + 2 + q, pb)
                   for q in (0, 1)]
            c_t = []
            for q in (0, 1):
                a_t[q].wait_recv()
                sbuf[l, 0, pl.ds(q * Q, Q)] = (
                    pt[q * Q:(q + 1) * Q]
                    + r1[l, 0, pl.ds(q * Q, Q)].astype(f32)).astype(bf16)
                c_t.append(start(sbuf.at[l, 0, pl.ds(q * Q, Q)],
                                 r2.at[l, 0, pl.ds(q * Q, Q)], s + 4 + q, pb))
            c_b = []
            for q in (0, 1):
                a_b[q].wait_recv()
                sbuf[l, 1, pl.ds(q * Q, Q)] = (
                    pbv[q * Q:(q + 1) * Q]
                    + r1[l, 1, pl.ds(q * Q, Q)].astype(f32)).astype(bf16)
                c_b.append(start(sbuf.at[l, 1, pl.ds(q * Q, Q)],
                                 r2.at[l, 1, pl.ds(q * Q, Q)], s + 6 + q, pa))
            c_t[0].wait_recv()
            c_t[1].wait_recv()
            if l < 2:
                xtop = (sbuf[l, 0].astype(f32)
                        + r2[l, 0].astype(f32)).astype(bf16)
            pend_bot = (c_b[0], c_b[1], l)

        out_ref[pl.ds(0, H), :] = sbuf[2, 0].astype(f32) + r2[2, 0].astype(f32)
        cb0, cb1, lp = pend_bot
        cb0.wait_recv()
        cb1.wait_recv()
        out_ref[pl.ds(H, H), :] = (sbuf[lp, 1].astype(f32)
                                   + r2[lp, 1].astype(f32))

        for rdma in pending_sends:
            rdma.wait_send()

    return pl.pallas_call(
        body,
        out_shape=jax.ShapeDtypeStruct((B, d), jnp.float32),
        in_specs=[pl.BlockSpec(memory_space=pltpu.VMEM)] * 7,
        out_specs=pl.BlockSpec(memory_space=pltpu.VMEM),
        scratch_shapes=[
            pltpu.VMEM((B, d), bf16),
            pltpu.VMEM((3, 2, H, d), bf16),
            pltpu.VMEM((3, 2, H, d), bf16),
            pltpu.VMEM((3, 2, H, d), bf16),
            pltpu.VMEM((3, 2, H, d), bf16),
            pltpu.SemaphoreType.DMA((27,)),
            pltpu.SemaphoreType.DMA((27,)),
        ],
        compiler_params=pltpu.CompilerParams(collective_id=0),
    )(x, Win0, Wout0, Win1, Wout1, Win2, Wout2)


# device time: 27172 ns/iter; 1.0221x vs baseline; 1.0221x over previous
import jax
import jax.numpy as jnp
from jax import lax
from jax.experimental import pallas as pl
from jax.experimental.pallas import tpu as pltpu

N_DEV = 4


def kernel(x, Win0, Wout0, Win1, Wout1, Win2, Wout2):
    m_per, d = x.shape
    B = N_DEV * m_per
    H = B // 2
    Q = H // 2

    f32 = jnp.float32
    bf16 = jnp.bfloat16

    def body(x_ref, win0, wout0, win1, wout1, win2, wout2,
             out_ref, xfull, pbuf, r1, sbuf, r2, send_sems, recv_sems):
        my_pos = lax.axis_index("i")
        pa = my_pos ^ 1
        pb = (N_DEV - 1) - my_pos
        pd = my_pos ^ 2

        barrier = pltpu.get_barrier_semaphore()
        for nbr in (pa, pb):
            pl.semaphore_signal(barrier, inc=1, device_id=(nbr,),
                                device_id_type=pl.DeviceIdType.MESH)
        pl.semaphore_wait(barrier, 2)

        pending_sends = []

        def start(src, dst, sem_idx, partner):
            rdma = pltpu.make_async_remote_copy(
                src_ref=src, dst_ref=dst,
                send_sem=send_sems.at[sem_idx],
                recv_sem=recv_sems.at[sem_idx],
                device_id=(partner,),
                device_id_type=pl.DeviceIdType.MESH,
            )
            rdma.start()
            pending_sends.append(rdma)
            return rdma

        row0 = my_pos * m_per
        xfull[pl.ds(row0, m_per), :] = x_ref[...].astype(bf16)
        my_rows = xfull.at[pl.ds(row0, m_per), :]
        for idx, peer in ((0, pa), (1, pb), (2, pd)):
            start(my_rows, my_rows, idx, peer)
        layers_bf = tuple(
            (win[...].astype(bf16), wout[...].astype(bf16))
            for win, wout in ((win0, wout0), (win1, wout1), (win2, wout2)))
        for idx, peer in ((0, pa), (1, pb), (2, pd)):
            recv = pltpu.make_async_remote_copy(
                src_ref=my_rows,
                dst_ref=xfull.at[pl.ds(peer * m_per, m_per), :],
                send_sem=send_sems.at[idx],
                recv_sem=recv_sems.at[idx],
                device_id=(peer,),
                device_id_type=pl.DeviceIdType.MESH,
            )
            recv.wait_recv()

        def qpart(xq, win_bf, wout_bf):
            h = jnp.maximum(
                jnp.dot(xq, win_bf, preferred_element_type=f32), 0.0)
            return jnp.dot(h.astype(bf16), wout_bf,
                           preferred_element_type=f32)

        st1_peer = (pa, pb)
        st2_peer = (pb, pa)
        xq = [[xfull[pl.ds((2 * h + q) * Q, Q), :] for q in (0, 1)]
              for h in (0, 1)]
        c_prev = None
        pq = [[None, None], [None, None]]
        for l, (win_bf, wout_bf) in enumerate(layers_bf):
            s = 3 + 8 * l
            a = [[None, None], [None, None]]
            for h in (0, 1):
                for q in (0, 1):
                    if c_prev is not None:
                        c_prev[h][q].wait_recv()
                        lp = l - 1
                        xq[h][q] = (sbuf[lp, h, pl.ds(q * Q, Q)].astype(f32)
                                    + r2[lp, h, pl.ds(q * Q, Q)].astype(f32)
                                    ).astype(bf16)
                    pq[h][q] = qpart(xq[h][q], win_bf, wout_bf)
                    pbuf[l, h, pl.ds(q * Q, Q)] = pq[h][q].astype(bf16)
                    a[h][q] = start(pbuf.at[l, h, pl.ds(q * Q, Q)],
                                    r1.at[l, h, pl.ds(q * Q, Q)],
                                    s + 2 * h + q, st1_peer[h])
            c = [[None, None], [None, None]]
            for h in (0, 1):
                for q in (0, 1):
                    a[h][q].wait_recv()
                    sbuf[l, h, pl.ds(q * Q, Q)] = (
                        pq[h][q]
                        + r1[l, h, pl.ds(q * Q, Q)].astype(f32)).astype(bf16)
                    c[h][q] = start(sbuf.at[l, h, pl.ds(q * Q, Q)],
                                    r2.at[l, h, pl.ds(q * Q, Q)],
                                    s + 4 + 2 * h + q, st2_peer[h])
            c_prev = c

        for h in (0, 1):
            for q in (0, 1):
                c_prev[h][q].wait_recv()
                out_ref[pl.ds((2 * h + q) * Q, Q), :] = (
                    sbuf[2, h, pl.ds(q * Q, Q)].astype(f32)
                    + r2[2, h, pl.ds(q * Q, Q)].astype(f32))

        for rdma in pending_sends:
            rdma.wait_send()

    return pl.pallas_call(
        body,
        out_shape=jax.ShapeDtypeStruct((B, d), jnp.float32),
        in_specs=[pl.BlockSpec(memory_space=pltpu.VMEM)] * 7,
        out_specs=pl.BlockSpec(memory_space=pltpu.VMEM),
        scratch_shapes=[
            pltpu.VMEM((B, d), bf16),
            pltpu.VMEM((3, 2, H, d), bf16),
            pltpu.VMEM((3, 2, H, d), bf16),
            pltpu.VMEM((3, 2, H, d), bf16),
            pltpu.VMEM((3, 2, H, d), bf16),
            pltpu.SemaphoreType.DMA((27,)),
            pltpu.SemaphoreType.DMA((27,)),
        ],
        compiler_params=pltpu.CompilerParams(collective_id=0),
    )(x, Win0, Wout0, Win1, Wout1, Win2, Wout2)


# device time: 26832 ns/iter; 1.0350x vs baseline; 1.0127x over previous
import jax
import jax.numpy as jnp
from jax import lax
from jax.experimental import pallas as pl
from jax.experimental.pallas import tpu as pltpu

N_DEV = 4


def kernel(x, Win0, Wout0, Win1, Wout1, Win2, Wout2):
    m_per, d = x.shape
    B = N_DEV * m_per
    H = B // 2
    Q = H // 2

    f32 = jnp.float32
    bf16 = jnp.bfloat16

    def body(x_ref, win0, wout0, win1, wout1, win2, wout2,
             out_ref, xfull, pbuf, r1, sbuf, r2, send_sems, recv_sems):
        my_pos = lax.axis_index("i")
        pa = my_pos ^ 1
        pb = (N_DEV - 1) - my_pos
        pd = my_pos ^ 2

        barrier = pltpu.get_barrier_semaphore()
        for nbr in (pa, pb):
            pl.semaphore_signal(barrier, inc=1, device_id=(nbr,),
                                device_id_type=pl.DeviceIdType.MESH)
        pl.semaphore_wait(barrier, 2)

        pending_sends = []

        def start(src, dst, sem_idx, partner, recv_sem_idx=None):
            rdma = pltpu.make_async_remote_copy(
                src_ref=src, dst_ref=dst,
                send_sem=send_sems.at[sem_idx],
                recv_sem=recv_sems.at[
                    sem_idx if recv_sem_idx is None else recv_sem_idx],
                device_id=(partner,),
                device_id_type=pl.DeviceIdType.MESH,
            )
            rdma.start()
            pending_sends.append(rdma)
            return rdma

        row0 = my_pos * m_per
        xfull[pl.ds(row0, m_per), :] = x_ref[...].astype(bf16)
        my_rows = xfull.at[pl.ds(row0, m_per), :]
        for idx, peer in ((0, pd), (1, pa), (2, pb)):
            start(my_rows, my_rows, idx, peer, recv_sem_idx=my_pos)
        layers_bf = tuple(
            (win[...].astype(bf16), wout[...].astype(bf16))
            for win, wout in ((win0, wout0), (win1, wout1), (win2, wout2)))

        def wait_chunk(j):
            @pl.when(j != my_pos)
            def _():
                recv = pltpu.make_async_remote_copy(
                    src_ref=my_rows,
                    dst_ref=xfull.at[pl.ds(j * m_per, m_per), :],
                    send_sem=send_sems.at[0],
                    recv_sem=recv_sems.at[j],
                    device_id=(pa,),
                    device_id_type=pl.DeviceIdType.MESH,
                )
                recv.wait_recv()

        def qpart(xq, win_bf, wout_bf):
            h = jnp.maximum(
                jnp.dot(xq, win_bf, preferred_element_type=f32), 0.0)
            return jnp.dot(h.astype(bf16), wout_bf,
                           preferred_element_type=f32)

        st1_peer = (pa, pb)
        st2_peer = (pb, pa)
        xq = [[None, None], [None, None]]
        c_prev = None
        pq = [[None, None], [None, None]]
        for l, (win_bf, wout_bf) in enumerate(layers_bf):
            s = 4 + 8 * l
            a = [[None, None], [None, None]]
            for h in (0, 1):
                for q in (0, 1):
                    if c_prev is None:
                        j = 2 * h + q
                        wait_chunk(j)
                        xq[h][q] = xfull[pl.ds(j * Q, Q), :]
                    else:
                        c_prev[h][q].wait_recv()
                        lp = l - 1
                        xq[h][q] = (sbuf[lp, h, pl.ds(q * Q, Q)].astype(f32)
                                    + r2[lp, h, pl.ds(q * Q, Q)].astype(f32)
                                    ).astype(bf16)
                    pq[h][q] = qpart(xq[h][q], win_bf, wout_bf)
                    pbuf[l, h, pl.ds(q * Q, Q)] = pq[h][q].astype(bf16)
                    a[h][q] = start(pbuf.at[l, h, pl.ds(q * Q, Q)],
                                    r1.at[l, h, pl.ds(q * Q, Q)],
                                    s + 2 * h + q, st1_peer[h])
            c = [[None, None], [None, None]]
            for h in (0, 1):
                for q in (0, 1):
                    a[h][q].wait_recv()
                    sbuf[l, h, pl.ds(q * Q, Q)] = (
                        pq[h][q]
                        + r1[l, h, pl.ds(q * Q, Q)].astype(f32)).astype(bf16)
                    c[h][q] = start(sbuf.at[l, h, pl.ds(q * Q, Q)],
                                    r2.at[l, h, pl.ds(q * Q, Q)],
                                    s + 4 + 2 * h + q, st2_peer[h])
            c_prev = c

        for h in (0, 1):
            for q in (0, 1):
                c_prev[h][q].wait_recv()
                out_ref[pl.ds((2 * h + q) * Q, Q), :] = (
                    sbuf[2, h, pl.ds(q * Q, Q)].astype(f32)
                    + r2[2, h, pl.ds(q * Q, Q)].astype(f32))

        for rdma in pending_sends:
            rdma.wait_send()

    return pl.pallas_call(
        body,
        out_shape=jax.ShapeDtypeStruct((B, d), jnp.float32),
        in_specs=[pl.BlockSpec(memory_space=pltpu.VMEM)] * 7,
        out_specs=pl.BlockSpec(memory_space=pltpu.VMEM),
        scratch_shapes=[
            pltpu.VMEM((B, d), bf16),
            pltpu.VMEM((3, 2, H, d), bf16),
            pltpu.VMEM((3, 2, H, d), bf16),
            pltpu.VMEM((3, 2, H, d), bf16),
            pltpu.VMEM((3, 2, H, d), bf16),
            pltpu.SemaphoreType.DMA((28,)),
            pltpu.SemaphoreType.DMA((28,)),
        ],
        compiler_params=pltpu.CompilerParams(collective_id=0),
    )(x, Win0, Wout0, Win1, Wout1, Win2, Wout2)
